# Initial kernel scaffold; baseline (speedup 1.0000x reference)
import jax
import jax.numpy as jnp
from jax.experimental import pallas as pl


def kernel(timestep, embeddings):
    raise NotImplementedError("write your pallas kernel here")


# speedup vs baseline: 5.2203x; 5.2203x over previous
import functools

import jax
import jax.numpy as jnp
from jax import lax
from jax.experimental import pallas as pl
from jax.experimental.pallas import tpu as pltpu
from jax.experimental.pallas import tpu_sc as plsc

EMB_DIM = 32
NC = 2
NS = 16
NW = NC * NS
IDX_PER_ROW = 128
K = 10
CH = K * IDX_PER_ROW
IDX_ROWS_W = 200
G = IDX_ROWS_W // K
PER_W = IDX_ROWS_W * IDX_PER_ROW
N_TOTAL = NW * PER_W

_mesh = plsc.VectorSubcoreMesh(core_axis_name="c", subcore_axis_name="s")


@functools.partial(
    pl.kernel,
    out_type=jax.ShapeDtypeStruct((N_TOTAL, EMB_DIM), jnp.float32),
    mesh=_mesh,
    scratch_types=[
        pltpu.VMEM((IDX_ROWS_W, IDX_PER_ROW), jnp.int32),
        pltpu.VMEM((CH, EMB_DIM), jnp.float32),
        pltpu.VMEM((CH, EMB_DIM), jnp.float32),
        pltpu.SemaphoreType.DMA,
        pltpu.SemaphoreType.DMA,
        pltpu.SemaphoreType.DMA,
    ],
    compiler_params=pltpu.CompilerParams(use_tc_tiling_on_sc=False),
)
def _sc_gather(idx_hbm, table_hbm, out_hbm, idx_v, rows0, rows1,
               gsem0, gsem1, wsem):
    wid = lax.axis_index("s") * NC + lax.axis_index("c")
    pltpu.sync_copy(idx_hbm.at[wid], idx_v)
    out_base = wid * PER_W

    def fire(sc, buf, sem):
        descs = []
        for j in range(K):
            d = pltpu.make_async_copy(
                table_hbm.at[idx_v.at[sc * K + j]],
                buf.at[pl.ds(j * IDX_PER_ROW, IDX_PER_ROW)],
                sem,
            )
            d.start()
            descs.append(d)
        return descs

    def drain(descs):
        for d in descs:
            d.wait()

    def wb(sc, buf):
        d = pltpu.make_async_copy(
            buf, out_hbm.at[pl.ds(out_base + sc * CH, CH)], wsem)
        d.start()
        return d

    def body(g, carry):
        sc0 = g * 2
        sc1 = sc0 + 1
        d0 = fire(sc0, rows0, gsem0)
        drain(d0)
        d1 = fire(sc1, rows1, gsem1)
        w0 = wb(sc0, rows0)
        drain(d1)
        w0.wait()
        w1 = wb(sc1, rows1)
        w1.wait()
        return carry

    lax.fori_loop(0, G // 2, body, 0)


def kernel(timestep, embeddings):
    idx = timestep.reshape(-1).astype(jnp.int32)
    idx = idx.reshape(NW, IDX_ROWS_W, IDX_PER_ROW)
    out = _sc_gather(idx, embeddings)
    return out.reshape(timestep.shape + (EMB_DIM,))


# speedup vs baseline: 5.2510x; 1.0059x over previous
import functools

import jax
import jax.numpy as jnp
from jax import lax
from jax.experimental import pallas as pl
from jax.experimental.pallas import tpu as pltpu
from jax.experimental.pallas import tpu_sc as plsc

EMB_DIM = 32
NC = 2
NS = 16
NW = NC * NS
PER_W = 25600
GSZ = 128
CH = 640
NG = CH // GSZ
G = PER_W // CH
NBUF = 4
IDX_ROWS = PER_W // GSZ
N_TOTAL = NW * PER_W

_mesh = plsc.VectorSubcoreMesh(core_axis_name="c", subcore_axis_name="s")


@functools.partial(
    pl.kernel,
    out_type=jax.ShapeDtypeStruct((N_TOTAL, EMB_DIM), jnp.float32),
    mesh=_mesh,
    scratch_types=[
        pltpu.VMEM((IDX_ROWS, GSZ), jnp.int32),
        [pltpu.VMEM((CH, EMB_DIM), jnp.float32) for _ in range(NBUF)],
        [pltpu.SemaphoreType.DMA for _ in range(NBUF)],
        [pltpu.SemaphoreType.DMA for _ in range(NBUF)],
    ],
    compiler_params=pltpu.CompilerParams(use_tc_tiling_on_sc=False),
)
def _sc_gather(idx_hbm, table_hbm, out_hbm, idx_v, bufs, gsems, wsems):
    wid = lax.axis_index("s") * NC + lax.axis_index("c")
    pltpu.sync_copy(idx_hbm.at[wid], idx_v)
    out_base = wid * PER_W

    def fire(c, b):
        for j in range(NG):
            pltpu.make_async_copy(
                table_hbm.at[idx_v.at[c * NG + j]],
                bufs[b].at[pl.ds(j * GSZ, GSZ)],
                gsems[b],
            ).start()

    def drain_g(b):
        pltpu.make_async_copy(
            out_hbm.at[pl.ds(0, CH)], bufs[b], gsems[b]).wait()

    def wb(c, b):
        pltpu.make_async_copy(
            bufs[b], out_hbm.at[pl.ds(out_base + c * CH, CH)], wsems[b]
        ).start()

    def drain_w(b):
        pltpu.make_async_copy(
            bufs[b], out_hbm.at[pl.ds(0, CH)], wsems[b]).wait()

    for b in range(NBUF):
        fire(b, b)

    nit = G // NBUF

    def body(i, carry):
        c0 = i * NBUF
        for b in range(NBUF):
            drain_g(b)
            wb(c0 + b, b)

        @pl.when(i + 1 < nit)
        def _():
            for b in range(NBUF):
                drain_w(b)
                fire(c0 + NBUF + b, b)

        return carry

    lax.fori_loop(0, nit, body, 0)
    for b in range(NBUF):
        drain_w(b)


def kernel(timestep, embeddings):
    idx = timestep.reshape(-1).astype(jnp.int32)
    idx = idx.reshape(NW, IDX_ROWS, GSZ)
    out = _sc_gather(idx, embeddings)
    return out.reshape(timestep.shape + (EMB_DIM,))
